# Initial kernel scaffold; baseline (speedup 1.0000x reference)
import jax
import jax.numpy as jnp
from jax.experimental import pallas as pl


def kernel(x, index, dim, dim_size):
    raise NotImplementedError("write your pallas kernel here")


# speedup vs baseline: 4.3721x; 4.3721x over previous
import functools

import jax
import jax.numpy as jnp
from jax import lax
from jax.experimental import pallas as pl
from jax.experimental.pallas import tpu as pltpu
from jax.experimental.pallas import tpu_sc as plsc

N_ROWS = 320000
D = 128
S = 10000
W = 128
NUM_WINDOWS = N_ROWS // W
NC, NS = 2, 16
NWORK = NC * NS
BASE_WIN = NUM_WINDOWS // NWORK
EXTRA = NUM_WINDOWS - BASE_WIN * NWORK
ROWS_PER_TILE_OUT = 624
_SLABS = (128, 128, 128, 128, 112)
_TAIL_BASE = NS * ROWS_PER_TILE_OUT
_TAIL = S - _TAIL_BASE
SCALE = 1.0 / (32.0 ** 0.5)


def _scatter_partials(x, idx):
    mesh = plsc.VectorSubcoreMesh(core_axis_name="c", subcore_axis_name="s")

    @functools.partial(
        pl.kernel,
        out_type=jax.ShapeDtypeStruct((NC * S, D), jnp.float32),
        mesh=mesh,
        scratch_types=[
            pltpu.VMEM((W,), jnp.int32),
            pltpu.VMEM((W, D), jnp.float32),
            pltpu.VMEM_SHARED((S, D), jnp.float32),
        ],
    )
    def k(x_hbm, idx_hbm, out_hbm, idx_v, rows_v, acc):
        c = lax.axis_index("c")
        s = lax.axis_index("s")
        wid = s * NC + c

        def zero_body(i, carry):
            r = i // (D // 16)
            col = (i % (D // 16)) * 16
            rows_v[r, pl.ds(col, 16)] = jnp.zeros((16,), jnp.float32)
            return carry

        lax.fori_loop(0, W * (D // 16), zero_body, 0)
        slab_base = s * ROWS_PER_TILE_OUT
        off = 0
        for size in _SLABS:
            pltpu.sync_copy(rows_v.at[pl.ds(0, size), :],
                            acc.at[pl.ds(slab_base + off, size), :])
            off += size

        @pl.when(s == 0)
        def _zero_tail():
            pltpu.sync_copy(rows_v.at[pl.ds(0, _TAIL), :],
                            acc.at[pl.ds(_TAIL_BASE, _TAIL), :])

        plsc.subcore_barrier()

        nw = jnp.where(wid < EXTRA, BASE_WIN + 1, BASE_WIN)

        def body(kk, carry):
            r0 = (wid + kk * NWORK) * W
            pltpu.sync_copy(idx_hbm.at[pl.ds(r0, W)], idx_v)
            pltpu.sync_copy(x_hbm.at[pl.ds(r0, W), :], rows_v)
            pltpu.sync_copy(rows_v, acc.at[idx_v], add=True)
            return carry

        lax.fori_loop(0, nw, body, 0)
        plsc.subcore_barrier()

        out_base = c * S + slab_base
        off = 0
        for size in _SLABS:
            pltpu.sync_copy(acc.at[pl.ds(slab_base + off, size), :],
                            out_hbm.at[pl.ds(out_base + off, size), :])
            off += size

        @pl.when(s == 0)
        def _write_tail():
            pltpu.sync_copy(acc.at[pl.ds(_TAIL_BASE, _TAIL), :],
                            out_hbm.at[pl.ds(c * S + _TAIL_BASE, _TAIL), :])

    return k(x, idx)


def _combine(partials):
    a = partials[:S]
    b = partials[S:]
    blk = S // 10

    def ck(a_ref, b_ref, o_ref):
        o_ref[...] = (a_ref[...] + b_ref[...]) * SCALE

    return pl.pallas_call(
        ck,
        grid=(10,),
        in_specs=[pl.BlockSpec((blk, D), lambda i: (i, 0)),
                  pl.BlockSpec((blk, D), lambda i: (i, 0))],
        out_specs=pl.BlockSpec((blk, D), lambda i: (i, 0)),
        out_shape=jax.ShapeDtypeStruct((S, D), jnp.float32),
    )(a, b)


def kernel(x, index, dim, dim_size):
    del dim, dim_size
    idx = index.astype(jnp.int32)
    partials = _scatter_partials(x, idx)
    return _combine(partials)


# speedup vs baseline: 6.6490x; 1.5208x over previous
import functools

import jax
import jax.numpy as jnp
from jax import lax
from jax.experimental import pallas as pl
from jax.experimental.pallas import tpu as pltpu
from jax.experimental.pallas import tpu_sc as plsc

N_ROWS = 320000
D = 128
S = 10000
W = 64
NUM_WINDOWS = N_ROWS // W
NC, NS = 2, 16
NWORK = NC * NS
BASE_WIN = NUM_WINDOWS // NWORK
EXTRA = NUM_WINDOWS - BASE_WIN * NWORK
ROWS_PER_TILE_OUT = 624
_SLABS = (64,) * 9 + (48,)
_TAIL_BASE = NS * ROWS_PER_TILE_OUT
_TAIL = S - _TAIL_BASE
SCALE = 1.0 / (32.0 ** 0.5)


NB = 4
LA = 2
T_OUT = BASE_WIN // NB


def _scatter_partials(x, idx):
    mesh = plsc.VectorSubcoreMesh(core_axis_name="c", subcore_axis_name="s")

    @functools.partial(
        pl.kernel,
        out_type=jax.ShapeDtypeStruct((NC * S, D), jnp.float32),
        mesh=mesh,
        scratch_types=[
            pltpu.VMEM((NB, W), jnp.int32),
            pltpu.VMEM((NB, W, D), jnp.float32),
            pltpu.VMEM((W, D), jnp.float32),
            pltpu.VMEM_SHARED((S, D), jnp.float32),
            pltpu.SemaphoreType.DMA((NB,)),
            pltpu.SemaphoreType.DMA((NB,)),
        ],
    )
    def k(x_hbm, idx_hbm, out_hbm, idx_v, rows_v, zbuf, acc, load_sem, scat_sem):
        c = lax.axis_index("c")
        s = lax.axis_index("s")
        wid = s * NC + c

        def zero_body(i, carry):
            r = i // (D // 16)
            col = (i % (D // 16)) * 16
            zbuf[r, pl.ds(col, 16)] = jnp.zeros((16,), jnp.float32)
            return carry

        lax.fori_loop(0, W * (D // 16), zero_body, 0)
        slab_base = s * ROWS_PER_TILE_OUT
        off = 0
        for size in _SLABS:
            pltpu.sync_copy(zbuf.at[pl.ds(0, size), :],
                            acc.at[pl.ds(slab_base + off, size), :])
            off += size

        @pl.when(s == 0)
        def _zero_tail():
            pltpu.sync_copy(zbuf.at[pl.ds(0, _TAIL), :],
                            acc.at[pl.ds(_TAIL_BASE, _TAIL), :])

        plsc.subcore_barrier()

        def _issue_load(b, w):
            pltpu.async_copy(idx_hbm.at[pl.ds(w * W, W)], idx_v.at[b],
                             load_sem.at[b])
            pltpu.async_copy(x_hbm.at[pl.ds(w * W, W), :], rows_v.at[b],
                             load_sem.at[b])

        def _wait_load(b, w):
            pltpu.make_async_copy(idx_hbm.at[pl.ds(w * W, W)], idx_v.at[b],
                                  load_sem.at[b]).wait()
            pltpu.make_async_copy(x_hbm.at[pl.ds(w * W, W), :], rows_v.at[b],
                                  load_sem.at[b]).wait()

        def _wait_scat(b):
            pltpu.make_async_copy(rows_v.at[b], acc.at[idx_v.at[b]],
                                  scat_sem.at[b]).wait()

        for b in range(LA):
            _issue_load(b, wid + b * NWORK)

        def outer(t, carry):
            for b in range(NB):
                j = t * NB + b
                w = wid + j * NWORK
                _wait_load(b, w)
                pltpu.async_copy(rows_v.at[b], acc.at[idx_v.at[b]],
                                 scat_sem.at[b], add=True)
                pb = (b + LA) % NB
                wp = wid + (j + LA) * NWORK
                if b < LA:
                    @pl.when(t > 0)
                    def _():
                        _wait_scat(pb)

                    _issue_load(pb, wp)
                else:
                    _wait_scat(pb)

                    @pl.when(t < T_OUT - 1)
                    def _():
                        _issue_load(pb, wp)
            return carry

        lax.fori_loop(0, T_OUT, outer, 0)
        for b in range(LA, NB):
            _wait_scat(b)

        @pl.when(wid < EXTRA)
        def _extra():
            w = wid + BASE_WIN * NWORK
            pltpu.sync_copy(idx_hbm.at[pl.ds(w * W, W)], idx_v.at[0])
            pltpu.sync_copy(x_hbm.at[pl.ds(w * W, W), :], rows_v.at[0])
            pltpu.sync_copy(rows_v.at[0], acc.at[idx_v.at[0]], add=True)

        plsc.subcore_barrier()

        out_base = c * S + slab_base
        pltpu.sync_copy(acc.at[pl.ds(slab_base, ROWS_PER_TILE_OUT), :],
                        out_hbm.at[pl.ds(out_base, ROWS_PER_TILE_OUT), :])

        @pl.when(s == 0)
        def _write_tail():
            pltpu.sync_copy(acc.at[pl.ds(_TAIL_BASE, _TAIL), :],
                            out_hbm.at[pl.ds(c * S + _TAIL_BASE, _TAIL), :])

    return k(x, idx)


def _combine(partials):
    a = partials[:S]
    b = partials[S:]
    blk = S // 10

    def ck(a_ref, b_ref, o_ref):
        o_ref[...] = (a_ref[...] + b_ref[...]) * SCALE

    return pl.pallas_call(
        ck,
        grid=(10,),
        in_specs=[pl.BlockSpec((blk, D), lambda i: (i, 0)),
                  pl.BlockSpec((blk, D), lambda i: (i, 0))],
        out_specs=pl.BlockSpec((blk, D), lambda i: (i, 0)),
        out_shape=jax.ShapeDtypeStruct((S, D), jnp.float32),
    )(a, b)


def kernel(x, index, dim, dim_size):
    del dim, dim_size
    idx = index.astype(jnp.int32)
    partials = _scatter_partials(x, idx)
    return _combine(partials)
